# baseline (speedup 1.0000x reference)
import dataclasses
import functools

import jax
import jax.numpy as jnp
from jax import lax
from jax.experimental import pallas as pl
from jax.experimental.pallas import tpu as pltpu
from jax.experimental.pallas import tpu_sc as plsc

NC = 2
NS = 16
NW = NC * NS
L = 16
CH = 128

_HIGH = lax.Precision.HIGHEST


def _mesh():
    return plsc.VectorSubcoreMesh(core_axis_name="c", subcore_axis_name="s")


def _sc_params():
    cp = pltpu.CompilerParams()
    if "needs_layout_passes" in pltpu.CompilerParams.__dataclass_fields__:
        cp = dataclasses.replace(cp, needs_layout_passes=False)
    return cp


def _pad(nn):
    return -(-nn // (NS * CH)) * (NS * CH)


def _deg_sc(dst, nn):
    E = dst.shape[0]
    EW = E // NW
    nnp = _pad(nn)
    assert E % (NW * L) == 0

    def body(dst_hbm, out_hbm, idx_v, acc_v):
        cid = lax.axis_index("c")
        sid = lax.axis_index("s")
        wid = sid * NC + cid
        pltpu.sync_copy(dst_hbm.at[pl.ds(wid * EW, EW)], idx_v)

        @pl.loop(0, nnp, step=L)
        def _(i):
            acc_v[0, pl.ds(i, L)] = jnp.zeros((L,), jnp.float32)

        ones = jnp.ones((L,), jnp.float32)
        z16 = jnp.zeros((L,), jnp.int32)

        @pl.loop(0, EW, step=L)
        def _(i):
            d16 = idx_v[pl.ds(i, L)]
            plsc.addupdate_scatter(acc_v, [z16, d16], ones)

        pltpu.sync_copy(acc_v, out_hbm.at[wid])

    k = pl.kernel(
        body,
        out_type=jax.ShapeDtypeStruct((NW, 1, nnp), jnp.float32),
        mesh=_mesh(),
        scratch_types=[
            pltpu.VMEM((EW,), jnp.int32),
            pltpu.VMEM((1, nnp), jnp.float32),
        ],
        compiler_params=_sc_params(),
    )
    return k(dst)


def _w_sc(src, dst, norm_row, nn):
    E = src.shape[0]
    EW = E // NW
    nnp = _pad(nn)
    assert E % (NW * L) == 0

    def body(src_hbm, dst_hbm, nr_hbm, out_hbm, sidx, didx, norm_v, w_v,
             sem):
        cid = lax.axis_index("c")
        sid = lax.axis_index("s")
        wid = sid * NC + cid
        base = wid * EW
        pltpu.async_copy(src_hbm.at[pl.ds(base, EW)], sidx, sem)
        pltpu.async_copy(dst_hbm.at[pl.ds(base, EW)], didx, sem)
        pltpu.async_copy(nr_hbm, norm_v, sem)

        @pl.loop(0, nnp, step=L)
        def _(i):
            w_v[0, pl.ds(i, L)] = jnp.zeros((L,), jnp.float32)

        pltpu.make_async_copy(src_hbm.at[pl.ds(0, EW)], sidx, sem).wait()
        pltpu.make_async_copy(dst_hbm.at[pl.ds(0, EW)], didx, sem).wait()
        pltpu.make_async_copy(nr_hbm, norm_v, sem).wait()

        z16 = jnp.zeros((L,), jnp.int32)

        @pl.loop(0, EW, step=L)
        def _(i):
            s16 = sidx[pl.ds(i, L)]
            d16 = didx[pl.ds(i, L)]
            vals = plsc.load_gather(norm_v, [z16, d16])
            plsc.addupdate_scatter(w_v, [z16, s16], vals)

        pltpu.sync_copy(w_v, out_hbm.at[wid])

    k = pl.kernel(
        body,
        out_type=jax.ShapeDtypeStruct((NW, 1, nnp), jnp.float32),
        mesh=_mesh(),
        scratch_types=[
            pltpu.VMEM((EW,), jnp.int32),
            pltpu.VMEM((EW,), jnp.int32),
            pltpu.VMEM((1, nnp), jnp.float32),
            pltpu.VMEM((1, nnp), jnp.float32),
            pltpu.SemaphoreType.DMA,
        ],
        compiler_params=_sc_params(),
    )
    return k(src, dst, norm_row)


def _agg_sc(hn, src_p, dst2, n_full):
    nn, D = hn.shape
    nnp = _pad(nn)
    RT = nnp // NS
    ZR = 32
    assert RT % ZR == 0 and n_full >= 6 and n_full % 4 == 0

    def body(hn_hbm, src_hbm, dst_hbm, agg_hbm,
             srci, didx, rows, agg_sh,
             semi0, semi1, semi2, semi3, semg0, semg1, sems0, sems1):
        semi = (semi0, semi1, semi2, semi3)
        semg = (semg0, semg1)
        sems = (sems0, sems1)
        cid = lax.axis_index("c")
        sid = lax.axis_index("s")
        wid = sid * NC + cid
        ebase = wid * n_full * CH

        pltpu.async_copy(dst_hbm.at[pl.ds(wid * n_full, n_full)], didx,
                         semg0)

        @pl.loop(0, ZR)
        def _(r):
            @pl.loop(0, D, step=L)
            def _(j):
                rows[0, r, pl.ds(j, L)] = jnp.zeros((L,), jnp.float32)

        @pl.loop(0, RT, step=ZR)
        def _(r):
            pltpu.async_copy(rows.at[0, pl.ds(0, ZR)],
                             agg_sh.at[pl.ds(sid * RT + r, ZR)], semg1)

        @pl.loop(0, RT, step=ZR)
        def _(r):
            pltpu.make_async_copy(rows.at[0, pl.ds(0, ZR)],
                                  agg_sh.at[pl.ds(0, ZR)], semg1).wait()

        pltpu.make_async_copy(dst_hbm.at[pl.ds(0, n_full)], didx,
                              semg0).wait()

        plsc.subcore_barrier()

        def idx_issue(c, s4):
            pltpu.async_copy(src_hbm.at[pl.ds(ebase + c * CH, CH)],
                             srci.at[s4], semi[s4])

        def idx_wait(s4):
            pltpu.make_async_copy(src_hbm.at[pl.ds(0, CH)],
                                  srci.at[s4], semi[s4]).wait()

        def gat_issue(s4, s2):
            pltpu.async_copy(hn_hbm.at[srci.at[s4]], rows.at[s2], semg[s2])

        def gat_wait(s2):
            pltpu.make_async_copy(
                hn_hbm.at[srci.at[0]], rows.at[s2], semg[s2]).wait()

        def sct_issue(c, s2):
            pltpu.async_copy(
                rows.at[s2], agg_sh.at[didx.at[c]], sems[s2], add=True)

        def sct_wait(s2):
            pltpu.make_async_copy(
                rows.at[s2], agg_sh.at[didx.at[0]], sems[s2]).wait()

        def step(c, s4, s2, do_wait2, do_prev, do_pref):
            if do_wait2:
                sct_wait(s2)
            idx_wait(s4)
            if do_prev:
                gat_wait(1 - s2)
                sct_issue(c - 1, 1 - s2)
            gat_issue(s4, s2)
            if do_pref:
                idx_issue(c + 2, (s4 + 2) % 4)

        idx_issue(0, 0)
        idx_issue(1, 1)
        for k4 in range(4):
            step(k4, k4, k4 % 2, k4 >= 2, k4 >= 1, True)

        @pl.loop(4, n_full - 4, step=4)
        def _(g):
            for k4 in range(4):
                step(g + k4, k4, k4 % 2, True, True, True)

        for k4 in range(4):
            c = n_full - 4 + k4
            step(c, k4, k4 % 2, True, True, c + 2 < n_full)

        gat_wait(1)
        sct_issue(n_full - 1, 1)
        sct_wait(0)
        sct_wait(1)

        plsc.subcore_barrier()

        @pl.loop(0, RT, step=ZR)
        def _(r):
            pltpu.async_copy(agg_sh.at[pl.ds(sid * RT + r, ZR)],
                             agg_hbm.at[cid, pl.ds(sid * RT + r, ZR)],
                             semg0)

        @pl.loop(0, RT, step=ZR)
        def _(r):
            pltpu.make_async_copy(agg_sh.at[pl.ds(0, ZR)],
                                  agg_hbm.at[cid, pl.ds(0, ZR)],
                                  semg0).wait()

    k = pl.kernel(
        body,
        out_type=jax.ShapeDtypeStruct((NC, nnp, D), jnp.float32),
        mesh=_mesh(),
        scratch_types=[
            pltpu.VMEM((4, CH), jnp.int32),
            pltpu.VMEM((n_full, CH), jnp.int32),
            pltpu.VMEM((2, CH, D), jnp.float32),
            pltpu.VMEM_SHARED((nnp, D), jnp.float32),
            pltpu.SemaphoreType.DMA,
            pltpu.SemaphoreType.DMA,
            pltpu.SemaphoreType.DMA,
            pltpu.SemaphoreType.DMA,
            pltpu.SemaphoreType.DMA,
            pltpu.SemaphoreType.DMA,
            pltpu.SemaphoreType.DMA,
            pltpu.SemaphoreType.DMA,
        ],
        compiler_params=_sc_params(),
    )
    return k(hn, src_p, dst2)


def _tc_linear(x, W_in, b_in):
    nn, D = x.shape

    def body(x_ref, w_ref, b_ref, h_ref):
        h_ref[...] = jnp.dot(
            x_ref[...], w_ref[...],
            preferred_element_type=jnp.float32, precision=_HIGH) + b_ref[...]

    return pl.pallas_call(
        body, out_shape=jax.ShapeDtypeStruct((nn, D), jnp.float32),
    )(x, W_in, b_in)


def _tc_norm(deg_part, h0):
    nn, D = h0.shape
    nnp = deg_part.shape[1]

    def body(dp_ref, h0_ref, h_ref, nc_ref, nr_ref):
        deg = jnp.sum(dp_ref[...], axis=0, keepdims=True)
        norm_r = lax.rsqrt(jnp.maximum(deg, 1.0))
        nr_ref[...] = norm_r
        norm_c = jnp.transpose(norm_r)
        nc_ref[...] = norm_c
        h_ref[...] = h0_ref[...] * norm_c[:nn]

    return pl.pallas_call(
        body,
        out_shape=[
            jax.ShapeDtypeStruct((nn, D), jnp.float32),
            jax.ShapeDtypeStruct((nnp, 1), jnp.float32),
            jax.ShapeDtypeStruct((1, nnp), jnp.float32),
        ],
    )(deg_part, h0)


def _tc_mid(agg_part, norm_col, W, b, nn):
    D = agg_part.shape[2]

    def body(a_ref, nc_ref, w_ref, b_ref, o_ref):
        norm = nc_ref[...][:nn]
        a = a_ref[...]
        agg = (a[0, :nn] + a[1, :nn]) * norm
        h = jnp.dot(agg, w_ref[...],
                    preferred_element_type=jnp.float32, precision=_HIGH)
        o_ref[...] = jnp.maximum(h + b_ref[...], 0.0) * norm

    return pl.pallas_call(
        body, out_shape=jax.ShapeDtypeStruct((nn, D), jnp.float32),
    )(agg_part, norm_col, W, b)


def _tc_final(agg_part, norm_col, norm_row, w_part, W1, b1, W2, b2,
              W_out, b_out, nn):
    D = agg_part.shape[2]

    def body(a_ref, nc_ref, nr_ref, wp_ref, w1_ref, b1_ref, w2_ref, b2_ref,
             wo_ref, bo_ref, o_ref):
        norm = nc_ref[...][:nn]
        a = a_ref[...]
        agg = (a[0, :nn] + a[1, :nn]) * norm
        hc = jnp.maximum(
            jnp.dot(agg, w1_ref[...],
                    preferred_element_type=jnp.float32, precision=_HIGH)
            + b1_ref[...], 0.0)
        w_row = jnp.sum(wp_ref[...], axis=0, keepdims=True)
        c_col = jnp.transpose(w_row * nr_ref[...])[:nn]
        v = jnp.sum(hc * c_col, axis=0, keepdims=True)
        t = jnp.dot(v, w2_ref[...],
                    preferred_element_type=jnp.float32, precision=_HIGH)
        t = t + jnp.float32(nn) * b2_ref[...]
        o_ref[...] = jnp.dot(t, wo_ref[...],
                             preferred_element_type=jnp.float32,
                             precision=_HIGH) + bo_ref[...]

    return pl.pallas_call(
        body, out_shape=jax.ShapeDtypeStruct((1, D), jnp.float32),
    )(agg_part, norm_col, norm_row, w_part, W1, b1, W2, b2, W_out, b_out)


def kernel(x, edge_index, W_in, b_in, W0, b0, W1, b1, W2, b2, W_out, b_out):
    nn, D = x.shape
    nnp = _pad(nn)
    b_in2 = b_in.reshape(1, D)
    b02 = b0.reshape(1, D)
    b12 = b1.reshape(1, D)
    b22 = b2.reshape(1, D)
    b_out2 = b_out.reshape(1, D)
    src = edge_index[0]
    dst = edge_index[1]

    E = src.shape[0]
    EW = E // NW
    n_full = -(-EW // CH)
    n_full = -(-n_full // 4) * 4
    EWp = n_full * CH
    assert E % NW == 0 and nn < nnp
    npad = EWp - EW
    pad_src = jnp.broadcast_to(jnp.arange(npad, dtype=src.dtype) % nn,
                               (NW, npad))
    pad_dst = jnp.broadcast_to(
        nn + jnp.arange(npad, dtype=dst.dtype) % (nnp - nn), (NW, npad))
    src_p = jnp.concatenate([src.reshape(NW, EW), pad_src],
                            axis=1).reshape(NW * EWp)
    dst2 = jnp.concatenate([dst.reshape(NW, EW), pad_dst],
                           axis=1).reshape(NW * n_full, CH)

    deg_part = _deg_sc(dst, nn).reshape(NW, nnp)
    h0 = _tc_linear(x, W_in, b_in2)
    h0n, norm_col, norm_row = _tc_norm(deg_part, h0)
    w_part = _w_sc(src, dst, norm_row, nn).reshape(NW, nnp)
    agg0 = _agg_sc(h0n, src_p, dst2, n_full)
    h1n = _tc_mid(agg0, norm_col, W0, b02, nn)
    agg1 = _agg_sc(h1n, src_p, dst2, n_full)
    return _tc_final(agg1, norm_col, norm_row, w_part, W1, b12, W2, b22,
                     W_out, b_out2, nn)


# speedup vs baseline: 1.0099x; 1.0099x over previous
import dataclasses
import functools

import jax
import jax.numpy as jnp
from jax import lax
from jax.experimental import pallas as pl
from jax.experimental.pallas import tpu as pltpu
from jax.experimental.pallas import tpu_sc as plsc

NC = 2
NS = 16
NW = NC * NS
L = 16
CH = 128

_HIGH = lax.Precision.HIGHEST


def _mesh():
    return plsc.VectorSubcoreMesh(core_axis_name="c", subcore_axis_name="s")


def _sc_params():
    cp = pltpu.CompilerParams()
    if "needs_layout_passes" in pltpu.CompilerParams.__dataclass_fields__:
        cp = dataclasses.replace(cp, needs_layout_passes=False)
    return cp


def _pad(nn):
    return -(-nn // (NS * CH)) * (NS * CH)


def _deg_sc(dst, nn):
    E = dst.shape[0]
    EW = E // NW
    nnp = _pad(nn)
    assert E % (NW * L) == 0

    def body(dst_hbm, out_hbm, idx_v, acc_v):
        cid = lax.axis_index("c")
        sid = lax.axis_index("s")
        wid = sid * NC + cid
        pltpu.sync_copy(dst_hbm.at[pl.ds(wid * EW, EW)], idx_v)

        @pl.loop(0, nnp, step=L)
        def _(i):
            acc_v[0, pl.ds(i, L)] = jnp.zeros((L,), jnp.float32)

        ones = jnp.ones((L,), jnp.float32)
        z16 = jnp.zeros((L,), jnp.int32)

        @pl.loop(0, EW, step=L)
        def _(i):
            d16 = idx_v[pl.ds(i, L)]
            plsc.addupdate_scatter(acc_v, [z16, d16], ones)

        pltpu.sync_copy(acc_v, out_hbm.at[wid])

    k = pl.kernel(
        body,
        out_type=jax.ShapeDtypeStruct((NW, 1, nnp), jnp.float32),
        mesh=_mesh(),
        scratch_types=[
            pltpu.VMEM((EW,), jnp.int32),
            pltpu.VMEM((1, nnp), jnp.float32),
        ],
        compiler_params=_sc_params(),
    )
    return k(dst)


def _w_sc(src, dst, norm_row, nn):
    E = src.shape[0]
    EW = E // NW
    nnp = _pad(nn)
    assert E % (NW * L) == 0

    def body(src_hbm, dst_hbm, nr_hbm, out_hbm, sidx, didx, norm_v, w_v,
             sem):
        cid = lax.axis_index("c")
        sid = lax.axis_index("s")
        wid = sid * NC + cid
        base = wid * EW
        pltpu.async_copy(src_hbm.at[pl.ds(base, EW)], sidx, sem)
        pltpu.async_copy(dst_hbm.at[pl.ds(base, EW)], didx, sem)
        pltpu.async_copy(nr_hbm, norm_v, sem)

        @pl.loop(0, nnp, step=L)
        def _(i):
            w_v[0, pl.ds(i, L)] = jnp.zeros((L,), jnp.float32)

        pltpu.make_async_copy(src_hbm.at[pl.ds(0, EW)], sidx, sem).wait()
        pltpu.make_async_copy(dst_hbm.at[pl.ds(0, EW)], didx, sem).wait()
        pltpu.make_async_copy(nr_hbm, norm_v, sem).wait()

        z16 = jnp.zeros((L,), jnp.int32)

        @pl.loop(0, EW, step=L)
        def _(i):
            s16 = sidx[pl.ds(i, L)]
            d16 = didx[pl.ds(i, L)]
            vals = plsc.load_gather(norm_v, [z16, d16])
            plsc.addupdate_scatter(w_v, [z16, s16], vals)

        pltpu.sync_copy(w_v, out_hbm.at[wid])

    k = pl.kernel(
        body,
        out_type=jax.ShapeDtypeStruct((NW, 1, nnp), jnp.float32),
        mesh=_mesh(),
        scratch_types=[
            pltpu.VMEM((EW,), jnp.int32),
            pltpu.VMEM((EW,), jnp.int32),
            pltpu.VMEM((1, nnp), jnp.float32),
            pltpu.VMEM((1, nnp), jnp.float32),
            pltpu.SemaphoreType.DMA,
        ],
        compiler_params=_sc_params(),
    )
    return k(src, dst, norm_row)


def _agg_sc(hn, src_p, dst2, n_full):
    nn, D = hn.shape
    nnp = _pad(nn)
    RT = nnp // NS
    ZR = 32
    assert RT % ZR == 0 and n_full >= 6 and n_full % 4 == 0

    def body(hn_hbm, src_hbm, dst_hbm, agg_hbm,
             srci, didx, rows, agg_sh,
             semi0, semi1, semi2, semi3, semg0, semg1, sems0, sems1):
        semi = (semi0, semi1, semi2, semi3)
        semg = (semg0, semg1)
        sems = (sems0, sems1)
        cid = lax.axis_index("c")
        sid = lax.axis_index("s")
        wid = sid * NC + cid
        ebase = wid * n_full * CH

        pltpu.async_copy(dst_hbm.at[pl.ds(wid * n_full, n_full)], didx,
                         semg0)

        @pl.loop(0, ZR)
        def _(r):
            @pl.loop(0, D, step=L)
            def _(j):
                rows[0, r, pl.ds(j, L)] = jnp.zeros((L,), jnp.float32)

        @pl.loop(0, RT, step=ZR)
        def _(r):
            pltpu.async_copy(rows.at[0, pl.ds(0, ZR)],
                             agg_sh.at[pl.ds(sid * RT + r, ZR)], semg1)

        @pl.loop(0, RT, step=ZR)
        def _(r):
            pltpu.make_async_copy(rows.at[0, pl.ds(0, ZR)],
                                  agg_sh.at[pl.ds(0, ZR)], semg1).wait()

        pltpu.make_async_copy(dst_hbm.at[pl.ds(0, n_full)], didx,
                              semg0).wait()

        plsc.subcore_barrier()

        def idx_issue(c, s4):
            pltpu.async_copy(src_hbm.at[pl.ds(ebase + c * CH, CH)],
                             srci.at[s4], semi[s4])

        def idx_wait(s4):
            pltpu.make_async_copy(src_hbm.at[pl.ds(0, CH)],
                                  srci.at[s4], semi[s4]).wait()

        def gat_issue(s4, s2):
            pltpu.async_copy(hn_hbm.at[srci.at[s4]], rows.at[s2], semg[s2])

        def gat_wait(s2):
            pltpu.make_async_copy(
                hn_hbm.at[srci.at[0]], rows.at[s2], semg[s2]).wait()

        def sct_issue(c, s2):
            pltpu.async_copy(
                rows.at[s2], agg_sh.at[didx.at[c]], sems[s2], add=True)

        def sct_wait(s2):
            pltpu.make_async_copy(
                rows.at[s2], agg_sh.at[didx.at[0]], sems[s2]).wait()

        def step(c, s4, s2, do_wait2, do_prev, do_pref):
            if do_wait2:
                sct_wait(s2)
            idx_wait(s4)
            if do_prev:
                gat_wait(1 - s2)
                sct_issue(c - 1, 1 - s2)
            gat_issue(s4, s2)
            if do_pref:
                idx_issue(c + 2, (s4 + 2) % 4)

        idx_issue(0, 0)
        idx_issue(1, 1)
        for k4 in range(4):
            step(k4, k4, k4 % 2, k4 >= 2, k4 >= 1, True)

        @pl.loop(4, n_full - 4, step=4)
        def _(g):
            for k4 in range(4):
                step(g + k4, k4, k4 % 2, True, True, True)

        for k4 in range(4):
            c = n_full - 4 + k4
            step(c, k4, k4 % 2, True, True, c + 2 < n_full)

        gat_wait(1)
        sct_issue(n_full - 1, 1)
        sct_wait(0)
        sct_wait(1)

        plsc.subcore_barrier()

        @pl.loop(0, RT, step=ZR)
        def _(r):
            pltpu.async_copy(agg_sh.at[pl.ds(sid * RT + r, ZR)],
                             agg_hbm.at[cid, pl.ds(sid * RT + r, ZR)],
                             semg0)

        @pl.loop(0, RT, step=ZR)
        def _(r):
            pltpu.make_async_copy(agg_sh.at[pl.ds(0, ZR)],
                                  agg_hbm.at[cid, pl.ds(0, ZR)],
                                  semg0).wait()

    k = pl.kernel(
        body,
        out_type=jax.ShapeDtypeStruct((NC, nnp, D), jnp.float32),
        mesh=_mesh(),
        scratch_types=[
            pltpu.VMEM((4, CH), jnp.int32),
            pltpu.VMEM((n_full, CH), jnp.int32),
            pltpu.VMEM((2, CH, D), jnp.float32),
            pltpu.VMEM_SHARED((nnp, D), jnp.float32),
            pltpu.SemaphoreType.DMA,
            pltpu.SemaphoreType.DMA,
            pltpu.SemaphoreType.DMA,
            pltpu.SemaphoreType.DMA,
            pltpu.SemaphoreType.DMA,
            pltpu.SemaphoreType.DMA,
            pltpu.SemaphoreType.DMA,
            pltpu.SemaphoreType.DMA,
        ],
        compiler_params=_sc_params(),
    )
    return k(hn, src_p, dst2)


def _tc_prolog(deg_part, x, W_in, b_in):
    nn, D = x.shape
    nnp = deg_part.shape[1]

    def body(dp_ref, x_ref, w_ref, b_ref, h_ref, nc_ref, nr_ref):
        deg = jnp.sum(dp_ref[...], axis=0, keepdims=True)
        norm_r = lax.rsqrt(jnp.maximum(deg, 1.0))
        nr_ref[...] = norm_r
        norm_c = jnp.transpose(norm_r)
        nc_ref[...] = norm_c
        h = jnp.dot(x_ref[...], w_ref[...],
                    preferred_element_type=jnp.float32, precision=_HIGH)
        h_ref[...] = (h + b_ref[...]) * norm_c[:nn]

    return pl.pallas_call(
        body,
        out_shape=[
            jax.ShapeDtypeStruct((nn, D), jnp.float32),
            jax.ShapeDtypeStruct((nnp, 1), jnp.float32),
            jax.ShapeDtypeStruct((1, nnp), jnp.float32),
        ],
    )(deg_part, x, W_in, b_in)


def _tc_mid(agg_part, norm_col, W, b, nn):
    D = agg_part.shape[2]

    def body(a_ref, nc_ref, w_ref, b_ref, o_ref):
        norm = nc_ref[...][:nn]
        a = a_ref[...]
        agg = (a[0, :nn] + a[1, :nn]) * norm
        h = jnp.dot(agg, w_ref[...],
                    preferred_element_type=jnp.float32, precision=_HIGH)
        o_ref[...] = jnp.maximum(h + b_ref[...], 0.0) * norm

    return pl.pallas_call(
        body, out_shape=jax.ShapeDtypeStruct((nn, D), jnp.float32),
    )(agg_part, norm_col, W, b)


def _tc_final(agg_part, norm_col, norm_row, w_part, W1, b1, W2, b2,
              W_out, b_out, nn):
    D = agg_part.shape[2]

    def body(a_ref, nc_ref, nr_ref, wp_ref, w1_ref, b1_ref, w2_ref, b2_ref,
             wo_ref, bo_ref, o_ref):
        norm = nc_ref[...][:nn]
        a = a_ref[...]
        agg = (a[0, :nn] + a[1, :nn]) * norm
        hc = jnp.maximum(
            jnp.dot(agg, w1_ref[...],
                    preferred_element_type=jnp.float32, precision=_HIGH)
            + b1_ref[...], 0.0)
        w_row = jnp.sum(wp_ref[...], axis=0, keepdims=True)
        c_col = jnp.transpose(w_row * nr_ref[...])[:nn]
        v = jnp.sum(hc * c_col, axis=0, keepdims=True)
        t = jnp.dot(v, w2_ref[...],
                    preferred_element_type=jnp.float32, precision=_HIGH)
        t = t + jnp.float32(nn) * b2_ref[...]
        o_ref[...] = jnp.dot(t, wo_ref[...],
                             preferred_element_type=jnp.float32,
                             precision=_HIGH) + bo_ref[...]

    return pl.pallas_call(
        body, out_shape=jax.ShapeDtypeStruct((1, D), jnp.float32),
    )(agg_part, norm_col, norm_row, w_part, W1, b1, W2, b2, W_out, b_out)


def kernel(x, edge_index, W_in, b_in, W0, b0, W1, b1, W2, b2, W_out, b_out):
    nn, D = x.shape
    nnp = _pad(nn)
    b_in2 = b_in.reshape(1, D)
    b02 = b0.reshape(1, D)
    b12 = b1.reshape(1, D)
    b22 = b2.reshape(1, D)
    b_out2 = b_out.reshape(1, D)
    src = edge_index[0]
    dst = edge_index[1]

    E = src.shape[0]
    EW = E // NW
    n_full = -(-EW // CH)
    n_full = -(-n_full // 4) * 4
    EWp = n_full * CH
    assert E % NW == 0 and nn < nnp
    npad = EWp - EW
    pad_src = jnp.broadcast_to(jnp.arange(npad, dtype=src.dtype) % nn,
                               (NW, npad))
    pad_dst = jnp.broadcast_to(
        nn + jnp.arange(npad, dtype=dst.dtype) % (nnp - nn), (NW, npad))
    src_p = jnp.concatenate([src.reshape(NW, EW), pad_src],
                            axis=1).reshape(NW * EWp)
    dst2 = jnp.concatenate([dst.reshape(NW, EW), pad_dst],
                           axis=1).reshape(NW * n_full, CH)

    deg_part = _deg_sc(dst, nn).reshape(NW, nnp)
    h0n, norm_col, norm_row = _tc_prolog(deg_part, x, W_in, b_in2)
    w_part = _w_sc(src, dst, norm_row, nn).reshape(NW, nnp)
    agg0 = _agg_sc(h0n, src_p, dst2, n_full)
    h1n = _tc_mid(agg0, norm_col, W0, b02, nn)
    agg1 = _agg_sc(h1n, src_p, dst2, n_full)
    return _tc_final(agg1, norm_col, norm_row, w_part, W1, b12, W2, b22,
                     W_out, b_out2, nn)


# speedup vs baseline: 1.0108x; 1.0009x over previous
import dataclasses

import jax
import jax.numpy as jnp
from jax import lax
from jax.experimental import pallas as pl
from jax.experimental.pallas import tpu as pltpu
from jax.experimental.pallas import tpu_sc as plsc

NC = 2
NS = 16
NW = NC * NS
L = 16
CH = 128

_HIGH = lax.Precision.HIGHEST


def _mesh():
    return plsc.VectorSubcoreMesh(core_axis_name="c", subcore_axis_name="s")


def _sc_params():
    cp = pltpu.CompilerParams()
    if "needs_layout_passes" in pltpu.CompilerParams.__dataclass_fields__:
        cp = dataclasses.replace(cp, needs_layout_passes=False)
    return cp


def _pad(nn):
    return -(-nn // (NS * CH)) * (NS * CH)


def _deg_sc(dst, nn):
    E = dst.shape[0]
    EW = E // NW
    nnp = _pad(nn)
    assert E % (NW * L) == 0

    def body(dst_hbm, out_hbm, idx_v, acc_v):
        cid = lax.axis_index("c")
        sid = lax.axis_index("s")
        wid = sid * NC + cid
        pltpu.sync_copy(dst_hbm.at[pl.ds(wid * EW, EW)], idx_v)

        @pl.loop(0, nnp, step=L)
        def _(i):
            acc_v[0, pl.ds(i, L)] = jnp.zeros((L,), jnp.float32)

        ones = jnp.ones((L,), jnp.float32)
        z16 = jnp.zeros((L,), jnp.int32)

        @pl.loop(0, EW, step=L)
        def _(i):
            d16 = idx_v[pl.ds(i, L)]
            plsc.addupdate_scatter(acc_v, [z16, d16], ones)

        pltpu.sync_copy(acc_v, out_hbm.at[wid])

    k = pl.kernel(
        body,
        out_type=jax.ShapeDtypeStruct((NW, 1, nnp), jnp.float32),
        mesh=_mesh(),
        scratch_types=[
            pltpu.VMEM((EW,), jnp.int32),
            pltpu.VMEM((1, nnp), jnp.float32),
        ],
        compiler_params=_sc_params(),
    )
    return k(dst)


def _w_sc(src, dst, norm_row, nn):
    E = src.shape[0]
    EW = E // NW
    nnp = _pad(nn)
    assert E % (NW * L) == 0

    def body(src_hbm, dst_hbm, nr_hbm, out_hbm, sidx, didx, norm_v, w_v,
             sem):
        cid = lax.axis_index("c")
        sid = lax.axis_index("s")
        wid = sid * NC + cid
        base = wid * EW
        pltpu.async_copy(src_hbm.at[pl.ds(base, EW)], sidx, sem)
        pltpu.async_copy(dst_hbm.at[pl.ds(base, EW)], didx, sem)
        pltpu.async_copy(nr_hbm, norm_v, sem)

        @pl.loop(0, nnp, step=L)
        def _(i):
            w_v[0, pl.ds(i, L)] = jnp.zeros((L,), jnp.float32)

        pltpu.make_async_copy(src_hbm.at[pl.ds(0, EW)], sidx, sem).wait()
        pltpu.make_async_copy(dst_hbm.at[pl.ds(0, EW)], didx, sem).wait()
        pltpu.make_async_copy(nr_hbm, norm_v, sem).wait()

        z16 = jnp.zeros((L,), jnp.int32)

        @pl.loop(0, EW, step=L)
        def _(i):
            s16 = sidx[pl.ds(i, L)]
            d16 = didx[pl.ds(i, L)]
            vals = plsc.load_gather(norm_v, [z16, d16])
            plsc.addupdate_scatter(w_v, [z16, s16], vals)

        pltpu.sync_copy(w_v, out_hbm.at[wid])

    k = pl.kernel(
        body,
        out_type=jax.ShapeDtypeStruct((NW, 1, nnp), jnp.float32),
        mesh=_mesh(),
        scratch_types=[
            pltpu.VMEM((EW,), jnp.int32),
            pltpu.VMEM((EW,), jnp.int32),
            pltpu.VMEM((1, nnp), jnp.float32),
            pltpu.VMEM((1, nnp), jnp.float32),
            pltpu.SemaphoreType.DMA,
        ],
        compiler_params=_sc_params(),
    )
    return k(src, dst, norm_row)


def _agg_sc(hn, src_p, dst2, n_full):
    nn, D = hn.shape
    nnp = _pad(nn)
    RT = nnp // NS
    ZR = 32
    assert RT % ZR == 0 and n_full >= 6 and n_full % 4 == 0

    def body(hn_hbm, src_hbm, dst_hbm, agg_hbm,
             srci, didx, rows, agg_sh,
             semi0, semi1, semi2, semi3, semg0, semg1, sems0, sems1):
        semi = (semi0, semi1, semi2, semi3)
        semg = (semg0, semg1)
        sems = (sems0, sems1)
        cid = lax.axis_index("c")
        sid = lax.axis_index("s")
        wid = sid * NC + cid
        ebase = wid * n_full * CH

        pltpu.async_copy(dst_hbm.at[pl.ds(wid * n_full, n_full)], didx,
                         semg0)

        @pl.loop(0, ZR)
        def _(r):
            @pl.loop(0, D, step=L)
            def _(j):
                rows[0, r, pl.ds(j, L)] = jnp.zeros((L,), jnp.float32)

        @pl.loop(0, RT, step=ZR)
        def _(r):
            pltpu.async_copy(rows.at[0, pl.ds(0, ZR)],
                             agg_sh.at[pl.ds(sid * RT + r, ZR)], semg1)

        @pl.loop(0, RT, step=ZR)
        def _(r):
            pltpu.make_async_copy(rows.at[0, pl.ds(0, ZR)],
                                  agg_sh.at[pl.ds(0, ZR)], semg1).wait()

        pltpu.make_async_copy(dst_hbm.at[pl.ds(0, n_full)], didx,
                              semg0).wait()

        plsc.subcore_barrier()

        def idx_issue(c, s4):
            pltpu.async_copy(src_hbm.at[pl.ds(ebase + c * CH, CH)],
                             srci.at[s4], semi[s4])

        def idx_wait(s4):
            pltpu.make_async_copy(src_hbm.at[pl.ds(0, CH)],
                                  srci.at[s4], semi[s4]).wait()

        def gat_issue(s4, s2):
            pltpu.async_copy(hn_hbm.at[srci.at[s4]], rows.at[s2], semg[s2])

        def gat_wait(s2):
            pltpu.make_async_copy(
                hn_hbm.at[srci.at[0]], rows.at[s2], semg[s2]).wait()

        def sct_issue(c, s2):
            pltpu.async_copy(
                rows.at[s2], agg_sh.at[didx.at[c]], sems[s2], add=True)

        def sct_wait(s2):
            pltpu.make_async_copy(
                rows.at[s2], agg_sh.at[didx.at[0]], sems[s2]).wait()

        def step(c, s4, s2, do_wait2, do_prev, do_pref):
            if do_wait2:
                sct_wait(s2)
            idx_wait(s4)
            if do_prev:
                gat_wait(1 - s2)
                sct_issue(c - 1, 1 - s2)
            gat_issue(s4, s2)
            if do_pref:
                idx_issue(c + 2, (s4 + 2) % 4)

        idx_issue(0, 0)
        idx_issue(1, 1)
        for k4 in range(4):
            step(k4, k4, k4 % 2, k4 >= 2, k4 >= 1, True)

        @pl.loop(4, n_full - 4, step=4)
        def _(g):
            for k4 in range(4):
                step(g + k4, k4, k4 % 2, True, True, True)

        for k4 in range(4):
            c = n_full - 4 + k4
            step(c, k4, k4 % 2, True, True, c + 2 < n_full)

        gat_wait(1)
        sct_issue(n_full - 1, 1)
        sct_wait(0)
        sct_wait(1)

        plsc.subcore_barrier()

        @pl.loop(0, RT, step=ZR)
        def _(r):
            pltpu.async_copy(agg_sh.at[pl.ds(sid * RT + r, ZR)],
                             agg_hbm.at[cid, pl.ds(sid * RT + r, ZR)],
                             semg0)

        @pl.loop(0, RT, step=ZR)
        def _(r):
            pltpu.make_async_copy(agg_sh.at[pl.ds(0, ZR)],
                                  agg_hbm.at[cid, pl.ds(0, ZR)],
                                  semg0).wait()

    k = pl.kernel(
        body,
        out_type=jax.ShapeDtypeStruct((NC, nnp, D), jnp.float32),
        mesh=_mesh(),
        scratch_types=[
            pltpu.VMEM((4, CH), jnp.int32),
            pltpu.VMEM((n_full, CH), jnp.int32),
            pltpu.VMEM((2, CH, D), jnp.float32),
            pltpu.VMEM_SHARED((nnp, D), jnp.float32),
            pltpu.SemaphoreType.DMA,
            pltpu.SemaphoreType.DMA,
            pltpu.SemaphoreType.DMA,
            pltpu.SemaphoreType.DMA,
            pltpu.SemaphoreType.DMA,
            pltpu.SemaphoreType.DMA,
            pltpu.SemaphoreType.DMA,
            pltpu.SemaphoreType.DMA,
        ],
        compiler_params=_sc_params(),
    )
    return k(hn, src_p, dst2)


def _tc_prolog(deg_part, x, W_in, b_in):
    nn, D = x.shape
    nnp = deg_part.shape[1]

    def body(dp_ref, x_ref, w_ref, b_ref, h_ref, nc_ref, nr_ref):
        deg = jnp.sum(dp_ref[...], axis=0, keepdims=True)
        norm_r = lax.rsqrt(jnp.maximum(deg, 1.0))
        nr_ref[...] = norm_r
        norm_c = jnp.transpose(norm_r)
        nc_ref[...] = norm_c
        h = jnp.dot(x_ref[...], w_ref[...],
                    preferred_element_type=jnp.float32, precision=_HIGH)
        h_ref[...] = (h + b_ref[...]) * norm_c[:nn]

    return pl.pallas_call(
        body,
        out_shape=[
            jax.ShapeDtypeStruct((nn, D), jnp.float32),
            jax.ShapeDtypeStruct((nnp, 1), jnp.float32),
            jax.ShapeDtypeStruct((1, nnp), jnp.float32),
        ],
    )(deg_part, x, W_in, b_in)


def _tc_mid(agg_part, norm_col, W, b, nn):
    D = agg_part.shape[2]

    def body(a_ref, nc_ref, w_ref, b_ref, o_ref):
        norm = nc_ref[...][:nn]
        a = a_ref[...]
        agg = (a[0, :nn] + a[1, :nn]) * norm
        h = jnp.dot(agg, w_ref[...],
                    preferred_element_type=jnp.float32, precision=_HIGH)
        o_ref[...] = jnp.maximum(h + b_ref[...], 0.0) * norm

    return pl.pallas_call(
        body, out_shape=jax.ShapeDtypeStruct((nn, D), jnp.float32),
    )(agg_part, norm_col, W, b)


def _tc_final(agg_part, norm_col, norm_row, w_part, W1, b1, W2, b2,
              W_out, b_out, nn):
    D = agg_part.shape[2]

    def body(a_ref, nc_ref, nr_ref, wp_ref, w1_ref, b1_ref, w2_ref, b2_ref,
             wo_ref, bo_ref, o_ref):
        norm = nc_ref[...][:nn]
        a = a_ref[...]
        agg = (a[0, :nn] + a[1, :nn]) * norm
        hc = jnp.maximum(
            jnp.dot(agg, w1_ref[...],
                    preferred_element_type=jnp.float32, precision=_HIGH)
            + b1_ref[...], 0.0)
        w_row = jnp.sum(wp_ref[...], axis=0, keepdims=True)
        c_col = jnp.transpose(w_row * nr_ref[...])[:nn]
        v = jnp.sum(hc * c_col, axis=0, keepdims=True)
        t = jnp.dot(v, w2_ref[...],
                    preferred_element_type=jnp.float32, precision=_HIGH)
        t = t + jnp.float32(nn) * b2_ref[...]
        o_ref[...] = jnp.dot(t, wo_ref[...],
                             preferred_element_type=jnp.float32,
                             precision=_HIGH) + bo_ref[...]

    return pl.pallas_call(
        body, out_shape=jax.ShapeDtypeStruct((1, D), jnp.float32),
    )(agg_part, norm_col, norm_row, w_part, W1, b1, W2, b2, W_out, b_out)


def kernel(x, edge_index, W_in, b_in, W0, b0, W1, b1, W2, b2, W_out, b_out):
    nn, D = x.shape
    nnp = _pad(nn)
    b_in2 = b_in.reshape(1, D)
    b02 = b0.reshape(1, D)
    b12 = b1.reshape(1, D)
    b22 = b2.reshape(1, D)
    b_out2 = b_out.reshape(1, D)
    src = edge_index[0]
    dst = edge_index[1]

    E = src.shape[0]
    EW = E // NW
    n_full = -(-EW // CH)
    n_full = -(-n_full // 4) * 4
    EWp = n_full * CH
    assert E % NW == 0 and nn < nnp
    npad = EWp - EW
    pad_src = jnp.broadcast_to(jnp.arange(npad, dtype=src.dtype) % nn,
                               (NW, npad))
    pad_dst = jnp.broadcast_to(
        nn + jnp.arange(npad, dtype=dst.dtype) % (nnp - nn), (NW, npad))
    src_p = jnp.concatenate([src.reshape(NW, EW), pad_src],
                            axis=1).reshape(NW * EWp)
    dst2 = jnp.concatenate([dst.reshape(NW, EW), pad_dst],
                           axis=1).reshape(NW * n_full, CH)

    deg_part = _deg_sc(dst, nn).reshape(NW, nnp)
    h0n, norm_col, norm_row = _tc_prolog(deg_part, x, W_in, b_in2)
    w_part = _w_sc(src, dst, norm_row, nn).reshape(NW, nnp)
    agg0 = _agg_sc(h0n, src_p, dst2, n_full)
    h1n = _tc_mid(agg0, norm_col, W0, b02, nn)
    agg1 = _agg_sc(h1n, src_p, dst2, n_full)
    return _tc_final(agg1, norm_col, norm_row, w_part, W1, b12, W2, b22,
                     W_out, b_out2, nn)
